# Initial kernel scaffold; baseline (speedup 1.0000x reference)
import jax
import jax.numpy as jnp
from jax.experimental import pallas as pl


def kernel(x, w1, b1, w2, b2, w3, b3, g1, beta1, g2, beta2):
    raise NotImplementedError("write your pallas kernel here")


# speedup vs baseline: 2.7293x; 2.7293x over previous
import functools

import jax
import jax.numpy as jnp
from jax.experimental import pallas as pl
from jax.experimental.pallas import tpu as pltpu

_BN_EPS = 1e-5
_N1, _N2, _N3 = 1024, 512, 256
_VMEM = 64 * 1024 * 1024


def _moments_kernel(x_ref, s_ref):
    t = pl.program_id(1)

    @pl.when(t == 0)
    def _():
        s_ref[...] = jnp.zeros_like(s_ref)

    xb = x_ref[...].astype(jnp.bfloat16)
    xa = jnp.concatenate(
        [xb, jnp.ones((8, xb.shape[1]), jnp.bfloat16)], axis=0)
    s_ref[0] += jax.lax.dot_general(
        xa, xa, (((1,), (1,)), ((), ())),
        preferred_element_type=jnp.float32)


def _mid_kernel(x_ref, s1_ref, w1f_ref, g1_ref, bt1_ref, w2_ref,
                y2_ref, ss_ref, sq_ref, w1s_ref, c1_ref, *, m_total, eps):
    t = pl.program_id(1)

    @pl.when(t == 0)
    def _():
        parts = s1_ref[0] + s1_ref[1]
        smat = parts[:32, :32]
        msum = parts[32:33, :32]
        w1f = w1f_ref[...]
        inv_m = 1.0 / m_total
        es = jnp.sum(w1f * (msum * inv_m), axis=1, keepdims=True)
        u = jax.lax.dot(w1f, smat,
                        precision=jax.lax.Precision.HIGHEST,
                        preferred_element_type=jnp.float32)
        q = jnp.sum(u * w1f, axis=1, keepdims=True)
        var = jnp.maximum(q * inv_m - es * es, 0.0)
        a1 = g1_ref[...] * jax.lax.rsqrt(var + eps)
        c1_ref[...] = bt1_ref[...] - a1 * es
        w1s_ref[...] = (w1f * a1).astype(jnp.bfloat16)
        ss_ref[...] = jnp.zeros_like(ss_ref)
        sq_ref[...] = jnp.zeros_like(sq_ref)

    xb = x_ref[...].astype(jnp.bfloat16)
    y1 = jnp.dot(w1s_ref[...], xb, preferred_element_type=jnp.float32)
    h1 = jnp.maximum(y1 + c1_ref[...], 0.0).astype(jnp.bfloat16)
    y2 = jnp.dot(w2_ref[...], h1, preferred_element_type=jnp.float32)
    y2_ref[...] = y2.astype(jnp.bfloat16)
    ss_ref[0] += jnp.sum(y2, axis=1, keepdims=True)
    sq_ref[0] += jnp.sum(y2 * y2, axis=1, keepdims=True)


def _out_kernel(y2_ref, s2_ref, q2_ref, g2_ref, bt2_ref, w3_ref, b3_ref,
                o_ref, *, m_total, eps):
    inv_m = 1.0 / m_total
    ssum = s2_ref[0] + s2_ref[1]
    sqs = q2_ref[0] + q2_ref[1]
    mean = ssum * inv_m
    var = jnp.maximum(sqs * inv_m - mean * mean, 0.0)
    a2 = g2_ref[...] * jax.lax.rsqrt(var + eps)
    c2 = bt2_ref[...] - a2 * mean
    h2 = jnp.maximum(y2_ref[...].astype(jnp.float32) * a2 + c2,
                     0.0).astype(jnp.bfloat16)
    o_ref[...] = (jnp.dot(w3_ref[...], h2, preferred_element_type=jnp.float32)
                  + b3_ref[...])


def kernel(x, w1, b1, w2, b2, w3, b3, g1, beta1, g2, beta2):
    B, Cin, L = x.shape
    M = B * L
    TL = 1024 if L % 1024 == 0 else L
    n_tiles = M // TL
    nT = n_tiles // 2
    t_per_b = L // TL
    nb = B // 2

    x2 = x.reshape(B * Cin, L)

    w1f = w1.T
    w2t = w2.T.astype(jnp.bfloat16)
    w3t = w3.T.astype(jnp.bfloat16)
    g1c = g1.reshape(_N1, 1)
    bt1c = beta1.reshape(_N1, 1)
    g2c = g2.reshape(_N2, 1)
    bt2c = beta2.reshape(_N2, 1)
    b3c = b3.reshape(_N3, 1)

    s1 = pl.pallas_call(
        _moments_kernel,
        out_shape=jax.ShapeDtypeStruct((2, 40, 40), jnp.float32),
        grid=(2, nb),
        in_specs=[pl.BlockSpec((Cin, L), lambda c, t: (c * nb + t, 0))],
        out_specs=pl.BlockSpec((1, 40, 40), lambda c, t: (c, 0, 0)),
        compiler_params=pltpu.CompilerParams(
            dimension_semantics=("parallel", "arbitrary"),
            vmem_limit_bytes=_VMEM),
    )(x2)

    y2, ss2, sq2 = pl.pallas_call(
        functools.partial(_mid_kernel, m_total=float(M), eps=_BN_EPS),
        out_shape=(jax.ShapeDtypeStruct((_N2, M), jnp.bfloat16),
                   jax.ShapeDtypeStruct((2, _N2, 1), jnp.float32),
                   jax.ShapeDtypeStruct((2, _N2, 1), jnp.float32)),
        grid=(2, nT),
        in_specs=[
            pl.BlockSpec((Cin, TL),
                         lambda c, t: ((c * nT + t) // t_per_b,
                                       (c * nT + t) % t_per_b)),
            pl.BlockSpec((2, 40, 40), lambda c, t: (0, 0, 0)),
            pl.BlockSpec((_N1, Cin), lambda c, t: (0, 0)),
            pl.BlockSpec((_N1, 1), lambda c, t: (0, 0)),
            pl.BlockSpec((_N1, 1), lambda c, t: (0, 0)),
            pl.BlockSpec((_N2, _N1), lambda c, t: (0, 0)),
        ],
        out_specs=(pl.BlockSpec((_N2, TL), lambda c, t: (0, c * nT + t)),
                   pl.BlockSpec((1, _N2, 1), lambda c, t: (c, 0, 0)),
                   pl.BlockSpec((1, _N2, 1), lambda c, t: (c, 0, 0))),
        scratch_shapes=[pltpu.VMEM((_N1, Cin), jnp.bfloat16),
                        pltpu.VMEM((_N1, 1), jnp.float32)],
        compiler_params=pltpu.CompilerParams(
            dimension_semantics=("parallel", "arbitrary"),
            vmem_limit_bytes=_VMEM),
    )(x2, s1, w1f, g1c, bt1c, w2t)

    o2 = pl.pallas_call(
        functools.partial(_out_kernel, m_total=float(M), eps=_BN_EPS),
        out_shape=jax.ShapeDtypeStruct((B * _N3, L), jnp.float32),
        grid=(2, nT),
        in_specs=[
            pl.BlockSpec((_N2, TL), lambda c, t: (0, c * nT + t)),
            pl.BlockSpec((2, _N2, 1), lambda c, t: (0, 0, 0)),
            pl.BlockSpec((2, _N2, 1), lambda c, t: (0, 0, 0)),
            pl.BlockSpec((_N2, 1), lambda c, t: (0, 0)),
            pl.BlockSpec((_N2, 1), lambda c, t: (0, 0)),
            pl.BlockSpec((_N3, _N2), lambda c, t: (0, 0)),
            pl.BlockSpec((_N3, 1), lambda c, t: (0, 0)),
        ],
        out_specs=pl.BlockSpec((_N3, TL),
                               lambda c, t: ((c * nT + t) // t_per_b,
                                             (c * nT + t) % t_per_b)),
        compiler_params=pltpu.CompilerParams(
            dimension_semantics=("parallel", "arbitrary"),
            vmem_limit_bytes=_VMEM),
    )(y2, ss2, sq2, g2c, bt2c, w3t, b3c)

    return o2.reshape(B, _N3, L)
